# Initial kernel scaffold; baseline (speedup 1.0000x reference)
import jax
import jax.numpy as jnp
from jax.experimental import pallas as pl


def kernel(x, edge_index, edge_type, basis1, coeff1, basis2, coeff2, dec_W, dec_b):
    raise NotImplementedError("write your pallas kernel here")


# speedup vs baseline: 13.8152x; 13.8152x over previous
import functools

import jax
import jax.numpy as jnp
from jax import lax
from jax.experimental import pallas as pl
from jax.experimental.pallas import tpu as pltpu
from jax.experimental.pallas import tpu_sc as plsc

_N = 10000
_E = 320000
_R = 4
_B = 8
_H = 128

_NC = 2
_NS = 16
_NW = _NC * _NS
_CH = 128
_CPT = 79
_EPT = _CPT * _CH
_EPAD = _EPT * _NW
_RPT = 632
_NSEG = _NS * _RPT
_TAIL = _RPT - 4 * _CH


_BLK = 400


def _wcat(basis_ref, coeff_ref):
    c = coeff_ref[...].astype(jnp.bfloat16).astype(jnp.float32)
    ws = []
    for r in range(_R):
        b0 = basis_ref[0].astype(jnp.bfloat16).astype(jnp.float32)
        w = c[r, 0] * b0
        for b in range(1, _B):
            bb = basis_ref[b].astype(jnp.bfloat16).astype(jnp.float32)
            w = w + c[r, b] * bb
        ws.append(w)
    return jnp.concatenate(ws, axis=1)


def _table_body(x_ref, basis_ref, coeff_ref, t_ref):
    t_ref[...] = jnp.dot(x_ref[...], _wcat(basis_ref, coeff_ref),
                         preferred_element_type=jnp.float32)


def _combine_body(a0_ref, a1_ref, d0_ref, d1_ref, basis_ref, coeff_ref,
                  h_ref, t_ref):
    deg = d0_ref[:, 0:1] + d1_ref[:, 0:1]
    h = jnp.maximum((a0_ref[...] + a1_ref[...]) / jnp.maximum(deg, 1.0), 0.0)
    h_ref[...] = h
    t_ref[...] = jnp.dot(h, _wcat(basis_ref, coeff_ref),
                         preferred_element_type=jnp.float32)


def _final_body(a0_ref, a1_ref, d0_ref, d1_ref, w_ref, b_ref, h_ref, r_ref):
    deg = d0_ref[:, 0:1] + d1_ref[:, 0:1]
    h = jnp.maximum((a0_ref[...] + a1_ref[...]) / jnp.maximum(deg, 1.0), 0.0)
    h_ref[...] = h
    r_ref[...] = jnp.dot(h, w_ref[...],
                         preferred_element_type=jnp.float32) + b_ref[0, 0]


def _tc_table(x, basis, coeff):
    return pl.pallas_call(
        _table_body,
        grid=(_N // _BLK,),
        in_specs=[
            pl.BlockSpec((_BLK, _H), lambda i: (i, 0)),
            pl.BlockSpec((_B, _H, _H), lambda i: (0, 0, 0)),
            pl.BlockSpec((_R, _B), lambda i: (0, 0)),
        ],
        out_specs=pl.BlockSpec((_BLK, _R * _H), lambda i: (i, 0)),
        out_shape=jax.ShapeDtypeStruct((_N, _R * _H), jnp.float32),
    )(x, basis, coeff)


def _tc_combine(a0, a1, d0, d1, basis, coeff):
    return pl.pallas_call(
        _combine_body,
        grid=(_N // _BLK,),
        in_specs=[
            pl.BlockSpec((_BLK, _H), lambda i: (i, 0)),
            pl.BlockSpec((_BLK, _H), lambda i: (i, 0)),
            pl.BlockSpec((_BLK, _H), lambda i: (i, 0)),
            pl.BlockSpec((_BLK, _H), lambda i: (i, 0)),
            pl.BlockSpec((_B, _H, _H), lambda i: (0, 0, 0)),
            pl.BlockSpec((_R, _B), lambda i: (0, 0)),
        ],
        out_specs=[
            pl.BlockSpec((_BLK, _H), lambda i: (i, 0)),
            pl.BlockSpec((_BLK, _R * _H), lambda i: (i, 0)),
        ],
        out_shape=[
            jax.ShapeDtypeStruct((_N, _H), jnp.float32),
            jax.ShapeDtypeStruct((_N, _R * _H), jnp.float32),
        ],
    )(a0, a1, d0, d1, basis, coeff)


def _tc_final(a0, a1, d0, d1, dec_w, dec_b):
    return pl.pallas_call(
        _final_body,
        grid=(_N // _BLK,),
        in_specs=[
            pl.BlockSpec((_BLK, _H), lambda i: (i, 0)),
            pl.BlockSpec((_BLK, _H), lambda i: (i, 0)),
            pl.BlockSpec((_BLK, _H), lambda i: (i, 0)),
            pl.BlockSpec((_BLK, _H), lambda i: (i, 0)),
            pl.BlockSpec((_H, 1), lambda i: (0, 0)),
            pl.BlockSpec((1, 1), lambda i: (0, 0)),
        ],
        out_specs=[
            pl.BlockSpec((_BLK, _H), lambda i: (i, 0)),
            pl.BlockSpec((_BLK, 1), lambda i: (i, 0)),
        ],
        out_shape=[
            jax.ShapeDtypeStruct((_N, _H), jnp.float32),
            jax.ShapeDtypeStruct((_N, 1), jnp.float32),
        ],
    )(a0, a1, d0, d1, dec_w, dec_b)


_SC_MESH = plsc.VectorSubcoreMesh(core_axis_name="c", subcore_axis_name="s")


def _sc_scatter_body(edge_hbm, tbl_hbm, agg_out,
                     pc_v, idxc_v, dstc_v, rows_v, agg_sp, sem):
    cid = lax.axis_index("c")
    sid = lax.axis_index("s")
    wid = cid * _NS + sid
    base = sid * _RPT

    def zero_row(i, _):
        for k in range(_H // 16):
            rows_v[i, pl.ds(k * 16, 16)] = jnp.zeros((16,), jnp.float32)
        return 0

    lax.fori_loop(0, _CH, zero_row, 0)

    for k in range(4):
        pltpu.sync_copy(rows_v, agg_sp.at[pl.ds(base + k * _CH, _CH)])
    pltpu.sync_copy(rows_v.at[pl.ds(0, _TAIL)],
                    agg_sp.at[pl.ds(base + 4 * _CH, _TAIL)])

    plsc.subcore_barrier()

    def chunk(j, _):
        pltpu.sync_copy(edge_hbm.at[wid, j], pc_v)
        for k in range(_CH // 16):
            p = pc_v[pl.ds(k * 16, 16)]
            dstc_v[pl.ds(k * 16, 16)] = lax.shift_right_logical(p, 16)
            idxc_v[pl.ds(k * 16, 16)] = lax.bitwise_and(
                p, jnp.full((16,), 0xFFFF, jnp.int32))
        pltpu.async_copy(tbl_hbm.at[idxc_v], rows_v, sem).wait()
        pltpu.sync_copy(rows_v, agg_sp.at[dstc_v], add=True)
        return 0

    lax.fori_loop(0, _CPT, chunk, 0)

    plsc.subcore_barrier()

    for k in range(4):
        s = pl.ds(base + k * _CH, _CH)
        pltpu.sync_copy(agg_sp.at[s], rows_v)
        pltpu.sync_copy(rows_v, agg_out.at[cid, s])
    t = pl.ds(base + 4 * _CH, _TAIL)
    pltpu.sync_copy(agg_sp.at[t], rows_v.at[pl.ds(0, _TAIL)])
    pltpu.sync_copy(rows_v.at[pl.ds(0, _TAIL)], agg_out.at[cid, t])


_sc_scatter = pl.kernel(
    _sc_scatter_body,
    out_type=[jax.ShapeDtypeStruct((_NC, _NSEG, _H), jnp.float32)],
    mesh=_SC_MESH,
    scratch_types=[
        pltpu.VMEM((_CH,), jnp.int32),
        pltpu.VMEM((_CH,), jnp.int32),
        pltpu.VMEM((_CH,), jnp.int32),
        pltpu.VMEM((_CH, _H), jnp.float32),
        pltpu.VMEM_SHARED((_NSEG, _H), jnp.float32),
        pltpu.SemaphoreType.DMA,
    ],
)


def _sc_deg_body(edge_hbm, deg_out, pc_v, dstc_v, ones_v, deg_sp, sem):
    cid = lax.axis_index("c")
    sid = lax.axis_index("s")
    wid = cid * _NS + sid
    base = sid * _RPT

    def zero_row(i, _):
        for k in range(_H // 16):
            ones_v[i, pl.ds(k * 16, 16)] = jnp.zeros((16,), jnp.float32)
        return 0

    lax.fori_loop(0, _CH, zero_row, 0)
    for k in range(4):
        pltpu.sync_copy(ones_v, deg_sp.at[pl.ds(base + k * _CH, _CH)])
    pltpu.sync_copy(ones_v.at[pl.ds(0, _TAIL)],
                    deg_sp.at[pl.ds(base + 4 * _CH, _TAIL)])

    def one_row(i, _):
        for k in range(_H // 16):
            ones_v[i, pl.ds(k * 16, 16)] = jnp.ones((16,), jnp.float32)
        return 0

    lax.fori_loop(0, _CH, one_row, 0)

    plsc.subcore_barrier()

    def chunk(j, _):
        pltpu.sync_copy(edge_hbm.at[wid, j], pc_v)
        for k in range(_CH // 16):
            p = pc_v[pl.ds(k * 16, 16)]
            dstc_v[pl.ds(k * 16, 16)] = lax.shift_right_logical(p, 16)
        pltpu.sync_copy(ones_v, deg_sp.at[dstc_v], add=True)
        return 0

    lax.fori_loop(0, _CPT, chunk, 0)

    plsc.subcore_barrier()

    for k in range(4):
        s = pl.ds(base + k * _CH, _CH)
        pltpu.sync_copy(deg_sp.at[s], ones_v)
        pltpu.sync_copy(ones_v, deg_out.at[cid, s])
    t = pl.ds(base + 4 * _CH, _TAIL)
    pltpu.sync_copy(deg_sp.at[t], ones_v.at[pl.ds(0, _TAIL)])
    pltpu.sync_copy(ones_v.at[pl.ds(0, _TAIL)], deg_out.at[cid, t])


_sc_deg = pl.kernel(
    _sc_deg_body,
    out_type=[jax.ShapeDtypeStruct((_NC, _NSEG, _H), jnp.float32)],
    mesh=_SC_MESH,
    scratch_types=[
        pltpu.VMEM((_CH,), jnp.int32),
        pltpu.VMEM((_CH,), jnp.int32),
        pltpu.VMEM((_CH, _H), jnp.float32),
        pltpu.VMEM_SHARED((_NSEG, _H), jnp.float32),
        pltpu.SemaphoreType.DMA,
    ],
)


def kernel(x, edge_index, edge_type, basis1, coeff1, basis2, coeff2,
           dec_W, dec_b):
    src = edge_index[0]
    dst = edge_index[1]
    idx = src * _R + edge_type
    pad = _EPAD - _E
    packed = jnp.concatenate(
        [(dst << 16) | idx,
         jnp.full((pad,), _N << 16, jnp.int32)]).reshape(_NW, _CPT, _CH)

    t1 = _tc_table(x, basis1, coeff1).reshape(_N * _R, _H)
    [deg_all] = _sc_deg(packed)
    [agg1] = _sc_scatter(packed, t1)
    h1, t2 = _tc_combine(agg1[0, :_N], agg1[1, :_N],
                         deg_all[0, :_N], deg_all[1, :_N], basis2, coeff2)
    [agg2] = _sc_scatter(packed, t2.reshape(_N * _R, _H))
    h2, rec = _tc_final(agg2[0, :_N], agg2[1, :_N],
                        deg_all[0, :_N], deg_all[1, :_N],
                        dec_W, dec_b.reshape(1, 1))
    return (rec[:, 0], h2)
